# baseline (speedup 1.0000x reference)
import functools

import jax
import jax.numpy as jnp
from jax.experimental import pallas as pl

_V = 100000
_K = 64
_LAMB = 0.01
_B = 256
_TV = 2000
_NT = _V // _TV


def _fused_lse_kernel(q_ref, m_ref, e_ref, acc_ref):
    i = pl.program_id(0)
    q = q_ref[...]
    e = e_ref[...]
    logits = jax.lax.dot_general(
        q, e, (((1,), (1,)), ((), ())),
        preferred_element_type=jnp.float32)
    ids = jax.lax.broadcasted_iota(jnp.int32, logits.shape, 1) + i * _TV
    masked = jnp.where(ids == m_ref[...], 0.0, jnp.exp(logits))
    part = jnp.sum(masked, axis=1, keepdims=True)

    @pl.when(i == 0)
    def _init():
        acc_ref[...] = jnp.zeros_like(acc_ref)

    acc_ref[...] += part


@functools.partial(jax.jit, static_argnames=())
def kernel(triples, emb):
    sub = triples[:, 0]
    pred = triples[:, 1]
    obj = triples[:, 2]

    s = jnp.take(emb, sub, axis=0)
    p = jnp.take(emb, pred, axis=0)
    o = jnp.take(emb, obj, axis=0)

    q_o = jnp.concatenate([s[:, 2] * p[:, 2], s[:, 1] * p[:, 1], s[:, 0] * p[:, 0]], axis=-1)
    q_s = jnp.concatenate([p[:, 0] * o[:, 2], p[:, 1] * o[:, 1], p[:, 2] * o[:, 0]], axis=-1)
    q = jnp.concatenate([q_o, q_s], axis=0)
    m = jnp.concatenate([obj, sub], axis=0)[:, None]
    e = emb.reshape(_V, 3 * _K)

    acc = pl.pallas_call(
        _fused_lse_kernel,
        grid=(_NT,),
        in_specs=[
            pl.BlockSpec((2 * _B, 3 * _K), lambda i: (0, 0)),
            pl.BlockSpec((2 * _B, 1), lambda i: (0, 0)),
            pl.BlockSpec((_TV, 3 * _K), lambda i: (i, 0)),
        ],
        out_specs=pl.BlockSpec((2 * _B, 1), lambda i: (0, 0)),
        out_shape=jax.ShapeDtypeStruct((2 * _B, 1), jnp.float32),
    )(q, m, e)

    lse = jnp.log(acc[:, 0])
    lse_o = lse[:_B]
    lse_s = lse[_B:]

    score = jnp.sum(s[:, 0] * p[:, 0] * o[:, 2]
                    + s[:, 1] * p[:, 1] * o[:, 1]
                    + s[:, 2] * p[:, 2] * o[:, 0], axis=-1)
    reg = (_LAMB / 3.0) * jnp.sum(jnp.abs(s) ** 3 + jnp.abs(p) ** 3 + jnp.abs(o) ** 3,
                                  axis=(1, 2))
    total_loss = jnp.sum(-2.0 * score + lse_o + lse_s + reg)
    return score, total_loss
